# baseline (speedup 1.0000x reference)
import functools

import jax
import jax.numpy as jnp
from jax import lax
from jax.experimental import pallas as pl
from jax.experimental.pallas import tpu as pltpu
from jax.experimental.pallas import tpu_sc as plsc

_B = 16384
_ED = 128
_NF = 8
_HID = 1024
_CAT = _NF * _ED

_NC = 2
_NS = 16
_NW = _NC * _NS
_CHUNK = 128

_NCH = 8
_CB = _B // _NCH
_BPW = _CB // _NW
_CHUNK = min(_CHUNK, _BPW)
_NGATH = _BPW // _CHUNK

_BB = 1024
_NBLK = _CB // _BB

_NBUF = 4


def _sc_gather_chunk(cats2d, tables, chunk):
  mesh = plsc.VectorSubcoreMesh(core_axis_name="c", subcore_axis_name="s")
  row0_chunk = chunk * (_CB // _CHUNK)

  @functools.partial(
      pl.kernel,
      out_type=jax.ShapeDtypeStruct((_CB, _CAT), jnp.float32),
      mesh=mesh,
      scratch_types=[
          pltpu.VMEM((_NF * _NGATH, _CHUNK), jnp.int32),
          pltpu.VMEM((_NBUF, _BPW, _ED), jnp.float32),
          pltpu.SemaphoreType.DMA,
          pltpu.SemaphoreType.DMA,
          pltpu.SemaphoreType.DMA,
      ],
  )
  def gather_kernel(c0, c1, c2, c3, c4, c5, c6, c7,
                    t0, t1, t2, t3, t4, t5, t6, t7,
                    out_hbm, idx_v, rows_v, sem_i, sem_g, sem_o):
    wid = lax.axis_index("s") * _NC + lax.axis_index("c")
    base = wid * _BPW
    row0 = row0_chunk + wid * _NGATH
    cat_refs = [c0, c1, c2, c3, c4, c5, c6, c7]
    tab_refs = [t0, t1, t2, t3, t4, t5, t6, t7]

    def gather_descr(f, c):
      return (tab_refs[f].at[idx_v.at[f * _NGATH + c]],
              rows_v.at[f % _NBUF].at[pl.ds(c * _CHUNK, _CHUNK)])

    def out_descr(f):
      return (rows_v.at[f % _NBUF],
              out_hbm.at[pl.ds(base, _BPW), pl.ds(f * _ED, _ED)])

    for f in range(_NF):
      pltpu.async_copy(cat_refs[f].at[pl.ds(row0, _NGATH)],
                       idx_v.at[pl.ds(f * _NGATH, _NGATH)], sem_i)
    for f in range(_NF):
      pltpu.make_async_copy(cat_refs[f].at[pl.ds(row0, _NGATH)],
                            idx_v.at[pl.ds(f * _NGATH, _NGATH)], sem_i).wait()

    for f in range(min(_NBUF, _NF)):
      for c in range(_NGATH):
        pltpu.async_copy(*gather_descr(f, c), sem_g)
    for f in range(_NF):
      for c in range(_NGATH):
        pltpu.make_async_copy(*gather_descr(f, c), sem_g).wait()
      pltpu.async_copy(*out_descr(f), sem_o)
      nxt = f + _NBUF
      if nxt < _NF:
        pltpu.make_async_copy(*out_descr(f), sem_o).wait()
        for c in range(_NGATH):
          pltpu.async_copy(*gather_descr(nxt, c), sem_g)
    for f in range(max(0, _NF - _NBUF), _NF):
      pltpu.make_async_copy(*out_descr(f), sem_o).wait()

  return gather_kernel(*cats2d, *tables)


def _mlp_body(emb_ref, price_ref, w1_ref, w1p_ref, b1_ref, w2_ref, b2_ref,
              *prev_and_out):
  out_ref = prev_and_out[-1]
  h = jnp.dot(emb_ref[...].astype(jnp.bfloat16), w1_ref[...],
              preferred_element_type=jnp.float32)
  h = h + price_ref[...] * w1p_ref[...] + b1_ref[...]
  h = jnp.maximum(h, 0.0)
  o = jnp.dot(h.astype(jnp.bfloat16), w2_ref[...],
              preferred_element_type=jnp.float32)
  o = jnp.maximum(o + b2_ref[...], 0.0)
  out_ref[...] = o


def _mlp_chunk(emb, price2d, w1a, w1p, b1, w2, b2, prev, chunk):
  blk0 = chunk * _NBLK
  in_specs = [
      pl.BlockSpec((_BB, _CAT), lambda i: (i, 0)),
      pl.BlockSpec((_BB, 1), lambda i: (i, 0)),
      pl.BlockSpec((_CAT, _HID), lambda i: (0, 0)),
      pl.BlockSpec((1, _HID), lambda i: (0, 0)),
      pl.BlockSpec((1, _HID), lambda i: (0, 0)),
      pl.BlockSpec((_HID, _HID), lambda i: (0, 0)),
      pl.BlockSpec((1, _HID), lambda i: (0, 0)),
  ]
  args = [emb, price2d, w1a, w1p, b1, w2, b2]
  aliases = {}
  if prev is not None:
    in_specs.append(pl.BlockSpec(memory_space=pl.ANY))
    args.append(prev)
    aliases = {7: 0}
  return pl.pallas_call(
      _mlp_body,
      grid=(_NBLK,),
      in_specs=in_specs,
      out_specs=pl.BlockSpec((_BB, _HID), lambda i: (blk0 + i, 0)),
      out_shape=jax.ShapeDtypeStruct((_B, _HID), jnp.float32),
      input_output_aliases=aliases,
      compiler_params=pltpu.CompilerParams(
          dimension_semantics=("arbitrary",),
      ),
  )(*args)


def kernel(cat_f0, cat_f1, cat_f2, cat_f3, cat_f4, cat_f5, cat_f6, cat_f7,
           x_price, E0, E1, E2, E3, E4, E5, E6, E7, W1, b1, W2, b2):
  cats = [cat_f0, cat_f1, cat_f2, cat_f3, cat_f4, cat_f5, cat_f6, cat_f7]
  tables = [E0, E1, E2, E3, E4, E5, E6, E7]
  cats2d = [c.reshape(_B // _CHUNK, _CHUNK) for c in cats]
  w1a = W1[:_CAT].astype(jnp.bfloat16)
  w1p = W1[_CAT:]
  b1r = b1[None, :]
  w2b = W2.astype(jnp.bfloat16)
  b2r = b2[None, :]
  price2d = x_price[:, None]

  embs = [_sc_gather_chunk(cats2d, tables, j) for j in range(_NCH)]
  out = None
  for j in range(_NCH):
    p = price2d[j * _CB:(j + 1) * _CB]
    out = _mlp_chunk(embs[j], p, w1a, w1p, b1r, w2b, b2r, out, j)
  return out


# speedup vs baseline: 1.2310x; 1.2310x over previous
import functools

import jax
import jax.numpy as jnp
from jax import lax
from jax.experimental import pallas as pl
from jax.experimental.pallas import tpu as pltpu
from jax.experimental.pallas import tpu_sc as plsc

_B = 16384
_ED = 128
_NF = 8
_HID = 1024
_CAT = _NF * _ED

_NC = 2
_NS = 16
_NW = _NC * _NS
_CHUNK = 128

_NCH = 4
_CB = _B // _NCH
_BPW = _CB // _NW
_CHUNK = min(_CHUNK, _BPW)
_NGATH = _BPW // _CHUNK

_BB = 1024
_NBLK = _CB // _BB

_NBUF = 4


def _sc_gather_chunk(cats2d, tables, chunk):
  mesh = plsc.VectorSubcoreMesh(core_axis_name="c", subcore_axis_name="s")
  row0_chunk = chunk * (_CB // _CHUNK)

  @functools.partial(
      pl.kernel,
      out_type=jax.ShapeDtypeStruct((_CB, _CAT), jnp.float32),
      mesh=mesh,
      scratch_types=[
          pltpu.VMEM((_NF * _NGATH, _CHUNK), jnp.int32),
          pltpu.VMEM((_NBUF, _BPW, _ED), jnp.float32),
          pltpu.SemaphoreType.DMA,
          pltpu.SemaphoreType.DMA,
          pltpu.SemaphoreType.DMA,
      ],
  )
  def gather_kernel(c0, c1, c2, c3, c4, c5, c6, c7,
                    t0, t1, t2, t3, t4, t5, t6, t7,
                    out_hbm, idx_v, rows_v, sem_i, sem_g, sem_o):
    wid = lax.axis_index("s") * _NC + lax.axis_index("c")
    base = wid * _BPW
    row0 = row0_chunk + wid * _NGATH
    cat_refs = [c0, c1, c2, c3, c4, c5, c6, c7]
    tab_refs = [t0, t1, t2, t3, t4, t5, t6, t7]

    def gather_descr(f, c):
      return (tab_refs[f].at[idx_v.at[f * _NGATH + c]],
              rows_v.at[f % _NBUF].at[pl.ds(c * _CHUNK, _CHUNK)])

    def out_descr(f):
      return (rows_v.at[f % _NBUF],
              out_hbm.at[pl.ds(base, _BPW), pl.ds(f * _ED, _ED)])

    for f in range(_NF):
      pltpu.async_copy(cat_refs[f].at[pl.ds(row0, _NGATH)],
                       idx_v.at[pl.ds(f * _NGATH, _NGATH)], sem_i)
    for f in range(_NF):
      pltpu.make_async_copy(cat_refs[f].at[pl.ds(row0, _NGATH)],
                            idx_v.at[pl.ds(f * _NGATH, _NGATH)], sem_i).wait()

    for f in range(min(_NBUF, _NF)):
      for c in range(_NGATH):
        pltpu.async_copy(*gather_descr(f, c), sem_g)
    for f in range(_NF):
      for c in range(_NGATH):
        pltpu.make_async_copy(*gather_descr(f, c), sem_g).wait()
      pltpu.async_copy(*out_descr(f), sem_o)
      nxt = f + _NBUF
      if nxt < _NF:
        pltpu.make_async_copy(*out_descr(f), sem_o).wait()
        for c in range(_NGATH):
          pltpu.async_copy(*gather_descr(nxt, c), sem_g)
    for f in range(max(0, _NF - _NBUF), _NF):
      pltpu.make_async_copy(*out_descr(f), sem_o).wait()

  return gather_kernel(*cats2d, *tables)


def _mlp_body(emb_ref, price_ref, w1_ref, w1p_ref, b1_ref, w2_ref, b2_ref,
              *prev_and_out):
  out_ref = prev_and_out[-1]
  h = jnp.dot(emb_ref[...].astype(jnp.bfloat16), w1_ref[...],
              preferred_element_type=jnp.float32)
  h = h + price_ref[...] * w1p_ref[...] + b1_ref[...]
  h = jnp.maximum(h, 0.0)
  o = jnp.dot(h.astype(jnp.bfloat16), w2_ref[...],
              preferred_element_type=jnp.float32)
  o = jnp.maximum(o + b2_ref[...], 0.0)
  out_ref[...] = o


def _mlp_chunk(emb, price2d, w1a, w1p, b1, w2, b2, prev, chunk):
  blk0 = chunk * _NBLK
  in_specs = [
      pl.BlockSpec((_BB, _CAT), lambda i: (i, 0)),
      pl.BlockSpec((_BB, 1), lambda i: (i, 0)),
      pl.BlockSpec((_CAT, _HID), lambda i: (0, 0)),
      pl.BlockSpec((1, _HID), lambda i: (0, 0)),
      pl.BlockSpec((1, _HID), lambda i: (0, 0)),
      pl.BlockSpec((_HID, _HID), lambda i: (0, 0)),
      pl.BlockSpec((1, _HID), lambda i: (0, 0)),
  ]
  args = [emb, price2d, w1a, w1p, b1, w2, b2]
  aliases = {}
  if prev is not None:
    in_specs.append(pl.BlockSpec(memory_space=pl.ANY))
    args.append(prev)
    aliases = {7: 0}
  return pl.pallas_call(
      _mlp_body,
      grid=(_NBLK,),
      in_specs=in_specs,
      out_specs=pl.BlockSpec((_BB, _HID), lambda i: (blk0 + i, 0)),
      out_shape=jax.ShapeDtypeStruct((_B, _HID), jnp.float32),
      input_output_aliases=aliases,
      compiler_params=pltpu.CompilerParams(
          dimension_semantics=("arbitrary",),
      ),
  )(*args)


def kernel(cat_f0, cat_f1, cat_f2, cat_f3, cat_f4, cat_f5, cat_f6, cat_f7,
           x_price, E0, E1, E2, E3, E4, E5, E6, E7, W1, b1, W2, b2):
  cats = [cat_f0, cat_f1, cat_f2, cat_f3, cat_f4, cat_f5, cat_f6, cat_f7]
  tables = [E0, E1, E2, E3, E4, E5, E6, E7]
  cats2d = [c.reshape(_B // _CHUNK, _CHUNK) for c in cats]
  w1a = W1[:_CAT].astype(jnp.bfloat16)
  w1p = W1[_CAT:]
  b1r = b1[None, :]
  w2b = W2.astype(jnp.bfloat16)
  b2r = b2[None, :]
  price2d = x_price[:, None]

  embs = [_sc_gather_chunk(cats2d, tables, j) for j in range(_NCH)]
  out = None
  for j in range(_NCH):
    p = price2d[j * _CB:(j + 1) * _CB]
    out = _mlp_chunk(embs[j], p, w1a, w1p, b1r, w2b, b2r, out, j)
  return out
